# Initial kernel scaffold; baseline (speedup 1.0000x reference)
import jax
import jax.numpy as jnp
from jax.experimental import pallas as pl


def kernel(input_ids, attention_mask, gumbel, emb, W):
    raise NotImplementedError("write your pallas kernel here")


# speedup vs baseline: 2.0023x; 2.0023x over previous
import functools

import jax
import jax.numpy as jnp
import numpy as np
from jax import lax
from jax.experimental import pallas as pl
from jax.experimental.pallas import tpu as pltpu
from jax.experimental.pallas import tpu_sc as plsc

_B, _S, _D, _N = 8, 64, 1024, 4096
_K = 1000
_EPS = float(np.finfo(np.float32).tiny)
_CN = 512
_NCH = _N // _CN
_NW = 32
_RPW = (_B * _S) // _NW


def _sc_gather(table, ids):
    mesh = plsc.VectorSubcoreMesh(core_axis_name="c", subcore_axis_name="s")

    @functools.partial(
        pl.kernel,
        out_type=jax.ShapeDtypeStruct((_B * _S, _D), jnp.float32),
        mesh=mesh,
        scratch_types=[
            pltpu.VMEM((_RPW,), jnp.int32),
            pltpu.VMEM((_RPW, _D), jnp.float32),
            pltpu.SemaphoreType.DMA,
        ],
    )
    def gather_kernel(table_hbm, ids_hbm, out_hbm, idx_v, rows_v, sem):
        wid = lax.axis_index("s") * 2 + lax.axis_index("c")
        base = wid * _RPW
        pltpu.sync_copy(ids_hbm.at[pl.ds(base, _RPW)], idx_v)
        pltpu.async_copy(table_hbm.at[idx_v], rows_v, sem).wait()
        pltpu.sync_copy(rows_v, out_hbm.at[pl.ds(base, _RPW)])

    return gather_kernel(table, ids)


def _tc_body(mask_ref, x_ref, gumbel_ref, w_ref, reps_ref, logp_ref,
             pooled_scr, reps_scr):
    j = pl.program_id(0)

    @pl.when(j == 0)
    def _pool():
        x = x_ref[...]
        m = mask_ref[...]
        mw = m[:, :, None]
        num = jnp.sum(x * mw, axis=1)
        den = jnp.maximum(jnp.sum(mw, axis=1), 1.0)
        pooled_scr[...] = num / den

    chunk = jnp.dot(pooled_scr[...], w_ref[...],
                    preferred_element_type=jnp.float32)
    reps_ref[...] = chunk
    reps_scr[:, pl.ds(j * _CN, _CN)] = chunk

    @pl.when(j == _NCH - 1)
    def _subset():
        reps = reps_scr[...]
        g = gumbel_ref[...]
        s0 = reps + g
        zero = jnp.zeros_like(s0)

        def body(_, carry):
            s, khot, onehot = carry
            s = s + jnp.log(jnp.maximum(1.0 - onehot, _EPS))
            e = jnp.exp(s - jnp.max(s, axis=-1, keepdims=True))
            onehot = e / jnp.sum(e, axis=-1, keepdims=True)
            return (s, khot + onehot, onehot)

        _, khot, _ = lax.fori_loop(0, _K, body, (s0, zero, zero))

        rmax = jnp.max(reps, axis=-1, keepdims=True)
        lse = rmax + jnp.log(jnp.sum(jnp.exp(reps - rmax), axis=-1,
                                     keepdims=True))
        lp = reps - lse

        ki = lax.bitcast_convert_type(khot, jnp.int32)

        def bs_val(_, c):
            lo, hi = c
            mid = lo + (hi - lo) // 2
            cnt = jnp.sum((ki >= mid).astype(jnp.int32), axis=-1,
                          keepdims=True)
            ge = cnt >= _K
            return (jnp.where(ge, mid, lo), jnp.where(ge, hi, mid))

        lo0 = jnp.zeros((_B, 1), jnp.int32)
        hi0 = jnp.full((_B, 1), 0x7F800000, jnp.int32)
        t, _ = lax.fori_loop(0, 31, bs_val, (lo0, hi0))

        gt = ki > t
        eq = ki == t
        n_gt = jnp.sum(gt.astype(jnp.int32), axis=-1, keepdims=True)
        need = _K - n_gt

        idx = lax.broadcasted_iota(jnp.int32, (_B, _N), 1)

        def bs_idx(_, c):
            lo, hi = c
            mid = lo + (hi - lo) // 2
            cnt = jnp.sum((eq & (idx <= mid)).astype(jnp.int32), axis=-1,
                          keepdims=True)
            ge = cnt >= need
            return (jnp.where(ge, lo, mid), jnp.where(ge, mid, hi))

        plo0 = jnp.full((_B, 1), -1, jnp.int32)
        phi0 = jnp.full((_B, 1), _N - 1, jnp.int32)
        _, p = lax.fori_loop(0, 13, bs_idx, (plo0, phi0))
        p = jnp.where(need > 0, p, -1)

        sel = gt | (eq & (idx <= p))
        w = (1.0 - khot) + khot
        logp = jnp.sum(jnp.where(sel, w * lp, 0.0), axis=-1, keepdims=True)
        logp_ref[...] = logp


def _tc_call(mask, x, gumbel, w):
    reps, logp = pl.pallas_call(
        _tc_body,
        grid=(_NCH,),
        in_specs=[
            pl.BlockSpec((_B, _S), lambda j: (0, 0)),
            pl.BlockSpec((_B, _S, _D), lambda j: (0, 0, 0)),
            pl.BlockSpec((_B, _N), lambda j: (0, 0)),
            pl.BlockSpec((_D, _CN), lambda j: (0, j)),
        ],
        out_specs=[
            pl.BlockSpec((_B, _CN), lambda j: (0, j)),
            pl.BlockSpec((_B, 1), lambda j: (0, 0)),
        ],
        out_shape=[
            jax.ShapeDtypeStruct((_B, _N), jnp.float32),
            jax.ShapeDtypeStruct((_B, 1), jnp.float32),
        ],
        scratch_shapes=[
            pltpu.VMEM((_B, _D), jnp.float32),
            pltpu.VMEM((_B, _N), jnp.float32),
        ],
    )(mask, x, gumbel, w)
    return reps, logp


def kernel(input_ids, attention_mask, gumbel, emb, W):
    ids = input_ids.reshape(-1).astype(jnp.int32)
    rows = _sc_gather(emb, ids)
    x = rows.reshape(_B, _S, _D)
    reps, logp = _tc_call(attention_mask, x, gumbel, W)
    return (logp.reshape(_B), reps)


# speedup vs baseline: 3.0832x; 1.5398x over previous
import functools

import jax
import jax.numpy as jnp
import numpy as np
from jax import lax
from jax.experimental import pallas as pl
from jax.experimental.pallas import tpu as pltpu
from jax.experimental.pallas import tpu_sc as plsc

_B, _S, _D, _N = 8, 64, 1024, 4096
_K = 1000
_EPS = float(np.finfo(np.float32).tiny)
_CN = 512
_NCH = _N // _CN
_NW = 32
_RPW = (_B * _S) // _NW


def _sc_gather(table, ids):
    mesh = plsc.VectorSubcoreMesh(core_axis_name="c", subcore_axis_name="s")

    @functools.partial(
        pl.kernel,
        out_type=jax.ShapeDtypeStruct((_B * _S, _D), jnp.float32),
        mesh=mesh,
        scratch_types=[
            pltpu.VMEM((_RPW,), jnp.int32),
            pltpu.VMEM((_RPW, _D), jnp.float32),
            pltpu.SemaphoreType.DMA,
        ],
    )
    def gather_kernel(table_hbm, ids_hbm, out_hbm, idx_v, rows_v, sem):
        wid = lax.axis_index("s") * 2 + lax.axis_index("c")
        base = wid * _RPW
        pltpu.sync_copy(ids_hbm.at[pl.ds(base, _RPW)], idx_v)
        pltpu.async_copy(table_hbm.at[idx_v], rows_v, sem).wait()
        pltpu.sync_copy(rows_v, out_hbm.at[pl.ds(base, _RPW)])

    return gather_kernel(table, ids)


def _tc_body(mask_ref, x_ref, gumbel_ref, w_ref, reps_ref, logp_ref,
             pooled_scr, reps_scr):
    j = pl.program_id(0)

    @pl.when(j == 0)
    def _pool():
        x = x_ref[...]
        m = mask_ref[...]
        mw = m[:, :, None]
        num = jnp.sum(x * mw, axis=1)
        den = jnp.maximum(jnp.sum(mw, axis=1), 1.0)
        pooled_scr[...] = num / den

    chunk = jnp.dot(pooled_scr[...], w_ref[...],
                    preferred_element_type=jnp.float32)
    reps_ref[...] = chunk
    reps_scr[:, pl.ds(j * _CN, _CN)] = chunk

    @pl.when(j == _NCH - 1)
    def _subset():
        reps = reps_scr[...]
        g = gumbel_ref[...]
        s0 = reps + g
        zero = jnp.zeros_like(s0)

        u0 = jnp.exp(s0 - jnp.max(s0, axis=-1, keepdims=True))

        def body(_, carry):
            u, khot = carry
            zinv = 1.0 / jnp.sum(u, axis=-1, keepdims=True)
            p = u * zinv
            u = u * jnp.maximum(1.0 - p, _EPS)
            return (u, khot + p)

        _, khot = lax.fori_loop(0, _K, body, (u0, zero))

        rmax = jnp.max(reps, axis=-1, keepdims=True)
        lse = rmax + jnp.log(jnp.sum(jnp.exp(reps - rmax), axis=-1,
                                     keepdims=True))
        lp = reps - lse

        ki = lax.bitcast_convert_type(khot, jnp.int32)

        def bs_val(_, c):
            lo, hi = c
            mid = lo + (hi - lo) // 2
            cnt = jnp.sum((ki >= mid).astype(jnp.int32), axis=-1,
                          keepdims=True)
            ge = cnt >= _K
            return (jnp.where(ge, mid, lo), jnp.where(ge, hi, mid))

        lo0 = jnp.zeros((_B, 1), jnp.int32)
        hi0 = jnp.full((_B, 1), 0x7F800000, jnp.int32)
        t, _ = lax.fori_loop(0, 31, bs_val, (lo0, hi0))

        gt = ki > t
        eq = ki == t
        n_gt = jnp.sum(gt.astype(jnp.int32), axis=-1, keepdims=True)
        need = _K - n_gt

        idx = lax.broadcasted_iota(jnp.int32, (_B, _N), 1)

        def bs_idx(_, c):
            lo, hi = c
            mid = lo + (hi - lo) // 2
            cnt = jnp.sum((eq & (idx <= mid)).astype(jnp.int32), axis=-1,
                          keepdims=True)
            ge = cnt >= need
            return (jnp.where(ge, lo, mid), jnp.where(ge, mid, hi))

        plo0 = jnp.full((_B, 1), -1, jnp.int32)
        phi0 = jnp.full((_B, 1), _N - 1, jnp.int32)
        _, p = lax.fori_loop(0, 13, bs_idx, (plo0, phi0))
        p = jnp.where(need > 0, p, -1)

        sel = gt | (eq & (idx <= p))
        w = (1.0 - khot) + khot
        logp = jnp.sum(jnp.where(sel, w * lp, 0.0), axis=-1, keepdims=True)
        logp_ref[...] = logp


def _tc_call(mask, x, gumbel, w):
    reps, logp = pl.pallas_call(
        _tc_body,
        grid=(_NCH,),
        in_specs=[
            pl.BlockSpec((_B, _S), lambda j: (0, 0)),
            pl.BlockSpec((_B, _S, _D), lambda j: (0, 0, 0)),
            pl.BlockSpec((_B, _N), lambda j: (0, 0)),
            pl.BlockSpec((_D, _CN), lambda j: (0, j)),
        ],
        out_specs=[
            pl.BlockSpec((_B, _CN), lambda j: (0, j)),
            pl.BlockSpec((_B, 1), lambda j: (0, 0)),
        ],
        out_shape=[
            jax.ShapeDtypeStruct((_B, _N), jnp.float32),
            jax.ShapeDtypeStruct((_B, 1), jnp.float32),
        ],
        scratch_shapes=[
            pltpu.VMEM((_B, _D), jnp.float32),
            pltpu.VMEM((_B, _N), jnp.float32),
        ],
    )(mask, x, gumbel, w)
    return reps, logp


def kernel(input_ids, attention_mask, gumbel, emb, W):
    ids = input_ids.reshape(-1).astype(jnp.int32)
    rows = _sc_gather(emb, ids)
    x = rows.reshape(_B, _S, _D)
    reps, logp = _tc_call(attention_mask, x, gumbel, W)
    return (logp.reshape(_B), reps)


# speedup vs baseline: 3.4090x; 1.1057x over previous
import functools

import jax
import jax.numpy as jnp
import numpy as np
from jax import lax
from jax.experimental import pallas as pl
from jax.experimental.pallas import tpu as pltpu
from jax.experimental.pallas import tpu_sc as plsc

_B, _S, _D, _N = 8, 64, 1024, 4096
_K = 1000
_EPS = float(np.finfo(np.float32).tiny)
_CN = 512
_NCH = _N // _CN
_NW = 32
_RPW = (_B * _S) // _NW


def _sc_gather(table, ids):
    mesh = plsc.VectorSubcoreMesh(core_axis_name="c", subcore_axis_name="s")

    @functools.partial(
        pl.kernel,
        out_type=jax.ShapeDtypeStruct((_B * _S, _D), jnp.float32),
        mesh=mesh,
        scratch_types=[
            pltpu.VMEM((_RPW,), jnp.int32),
            pltpu.VMEM((_RPW, _D), jnp.float32),
            pltpu.SemaphoreType.DMA,
        ],
    )
    def gather_kernel(table_hbm, ids_hbm, out_hbm, idx_v, rows_v, sem):
        wid = lax.axis_index("s") * 2 + lax.axis_index("c")
        base = wid * _RPW
        pltpu.sync_copy(ids_hbm.at[pl.ds(base, _RPW)], idx_v)
        pltpu.async_copy(table_hbm.at[idx_v], rows_v, sem).wait()
        pltpu.sync_copy(rows_v, out_hbm.at[pl.ds(base, _RPW)])

    return gather_kernel(table, ids)


def _tc_body(mask_ref, x_ref, gumbel_ref, w_ref, reps_ref, logp_ref,
             pooled_scr, reps_scr, khot_scr):
    j = pl.program_id(0)

    @pl.when(j == 0)
    def _pool():
        x = x_ref[...]
        m = mask_ref[...]
        mw = m[:, :, None]
        num = jnp.sum(x * mw, axis=1)
        den = jnp.maximum(jnp.sum(mw, axis=1), 1.0)
        pooled_scr[...] = num / den

    chunk = jnp.dot(pooled_scr[...], w_ref[...],
                    preferred_element_type=jnp.float32)
    reps_ref[...] = chunk
    reps_scr[:, pl.ds(j * _CN, _CN)] = chunk

    @pl.when(j == _NCH - 1)
    def _subset():
        reps = reps_scr[...]
        g = gumbel_ref[...]
        s0 = reps + g

        u0 = jnp.exp(s0 - jnp.max(s0, axis=-1, keepdims=True))
        khot_scr[...] = jnp.zeros_like(s0)

        def body(_, u):
            zinv = 1.0 / jnp.sum(u, axis=-1, keepdims=True)
            p = u * zinv
            khot_scr[...] += p
            return u * jnp.maximum(1.0 - p, _EPS)

        lax.fori_loop(0, _K, body, u0)
        khot = khot_scr[...]

        rmax = jnp.max(reps, axis=-1, keepdims=True)
        lse = rmax + jnp.log(jnp.sum(jnp.exp(reps - rmax), axis=-1,
                                     keepdims=True))
        lp = reps - lse

        ki = lax.bitcast_convert_type(khot, jnp.int32)

        def bs_val(_, c):
            lo, hi = c
            mid = lo + (hi - lo) // 2
            cnt = jnp.sum((ki >= mid).astype(jnp.int32), axis=-1,
                          keepdims=True)
            ge = cnt >= _K
            return (jnp.where(ge, mid, lo), jnp.where(ge, hi, mid))

        lo0 = jnp.zeros((_B, 1), jnp.int32)
        hi0 = jnp.full((_B, 1), 0x7F800000, jnp.int32)
        t, _ = lax.fori_loop(0, 31, bs_val, (lo0, hi0))

        gt = ki > t
        eq = ki == t
        n_gt = jnp.sum(gt.astype(jnp.int32), axis=-1, keepdims=True)
        need = _K - n_gt

        idx = lax.broadcasted_iota(jnp.int32, (_B, _N), 1)

        def bs_idx(_, c):
            lo, hi = c
            mid = lo + (hi - lo) // 2
            cnt = jnp.sum((eq & (idx <= mid)).astype(jnp.int32), axis=-1,
                          keepdims=True)
            ge = cnt >= need
            return (jnp.where(ge, lo, mid), jnp.where(ge, mid, hi))

        plo0 = jnp.full((_B, 1), -1, jnp.int32)
        phi0 = jnp.full((_B, 1), _N - 1, jnp.int32)
        _, p = lax.fori_loop(0, 13, bs_idx, (plo0, phi0))
        p = jnp.where(need > 0, p, -1)

        sel = gt | (eq & (idx <= p))
        w = (1.0 - khot) + khot
        logp = jnp.sum(jnp.where(sel, w * lp, 0.0), axis=-1, keepdims=True)
        logp_ref[...] = logp


def _tc_call(mask, x, gumbel, w):
    reps, logp = pl.pallas_call(
        _tc_body,
        grid=(_NCH,),
        in_specs=[
            pl.BlockSpec((_B, _S), lambda j: (0, 0)),
            pl.BlockSpec((_B, _S, _D), lambda j: (0, 0, 0)),
            pl.BlockSpec((_B, _N), lambda j: (0, 0)),
            pl.BlockSpec((_D, _CN), lambda j: (0, j)),
        ],
        out_specs=[
            pl.BlockSpec((_B, _CN), lambda j: (0, j)),
            pl.BlockSpec((_B, 1), lambda j: (0, 0)),
        ],
        out_shape=[
            jax.ShapeDtypeStruct((_B, _N), jnp.float32),
            jax.ShapeDtypeStruct((_B, 1), jnp.float32),
        ],
        scratch_shapes=[
            pltpu.VMEM((_B, _D), jnp.float32),
            pltpu.VMEM((_B, _N), jnp.float32),
            pltpu.VMEM((_B, _N), jnp.float32),
        ],
    )(mask, x, gumbel, w)
    return reps, logp


def kernel(input_ids, attention_mask, gumbel, emb, W):
    ids = input_ids.reshape(-1).astype(jnp.int32)
    rows = _sc_gather(emb, ids)
    x = rows.reshape(_B, _S, _D)
    reps, logp = _tc_call(attention_mask, x, gumbel, W)
    return (logp.reshape(_B), reps)
